# Initial kernel scaffold; baseline (speedup 1.0000x reference)
import functools

import jax
import jax.numpy as jnp
from jax import lax
from jax.experimental import pallas as pl
from jax.experimental.pallas import tpu as pltpu
from jax.experimental.pallas import tpu_sc as plsc

_B = 512
_S = 14
_C = 30
_ROWS = _B * _S * _S
_NTILES = 32
_ROWS_PER_TILE = _ROWS // _NTILES
_CHUNK_ROWS = 784
_CHUNKS = _ROWS_PER_TILE // _CHUNK_ROWS
_CHUNK_WORDS = _CHUNK_ROWS * _C
_GROUPS = _CHUNK_ROWS // 16
_HALF = 0.5 * _S


def _sqrt16(x):
    i = plsc.bitcast(x, jnp.int32)
    i = jnp.int32(0x5F3759DF) - lax.shift_right_logical(i, 1)
    r = plsc.bitcast(i, jnp.float32)
    r = r * (1.5 - 0.5 * x * r * r)
    r = r * (1.5 - 0.5 * x * r * r)
    r = r * (1.5 - 0.5 * x * r * r)
    return x * r


def _sc_partials(p_flat, g_flat):
    mesh = plsc.VectorSubcoreMesh(core_axis_name="c", subcore_axis_name="s")

    @functools.partial(
        pl.kernel,
        mesh=mesh,
        out_type=jax.ShapeDtypeStruct((_NTILES, 16), jnp.float32),
        scratch_types=[
            pltpu.VMEM((_CHUNK_WORDS,), jnp.float32),
            pltpu.VMEM((_CHUNK_WORDS,), jnp.float32),
            pltpu.VMEM((16,), jnp.float32),
        ],
    )
    def body(p_hbm, g_hbm, out_hbm, pbuf, gbuf, acc):
        wid = lax.axis_index("s") * 2 + lax.axis_index("c")
        acc[...] = jnp.zeros((16,), jnp.float32)
        row_off = lax.iota(jnp.int32, 16) * _C
        tile_base = wid * (_ROWS_PER_TILE * _C)

        @pl.loop(0, _CHUNKS)
        def _chunk(k):
            off = tile_base + k * _CHUNK_WORDS
            pltpu.sync_copy(p_hbm.at[pl.ds(off, _CHUNK_WORDS)], pbuf)
            pltpu.sync_copy(g_hbm.at[pl.ds(off, _CHUNK_WORDS)], gbuf)

            @pl.loop(0, _GROUPS)
            def _group(gi):
                idx0 = row_off + gi * (16 * _C)

                def lp(c):
                    return plsc.load_gather(pbuf, [idx0 + c])

                def lg(c):
                    return plsc.load_gather(gbuf, [idx0 + c])

                px1, py1, pw1, ph1, pc1 = lp(0), lp(1), lp(2), lp(3), lp(4)
                px2, py2, pw2, ph2, pc2 = lp(5), lp(6), lp(7), lp(8), lp(9)
                gx, gy, gw, gh, g4 = lg(0), lg(1), lg(2), lg(3), lg(4)
                g9 = lg(9)

                cls = jnp.zeros((16,), jnp.float32)
                for c in range(10, _C):
                    dcv = lp(c) - lg(c)
                    cls = cls + dcv * dcv

                gltx = gx - _HALF * gw
                grbx = gx + _HALF * gw
                glty = gy - _HALF * gh
                grby = gy + _HALF * gh
                ag = (grbx - gltx) * (grby - glty)

                def iou(px, py, pw, ph):
                    pltx = px - _HALF * pw
                    prbx = px + _HALF * pw
                    plty = py - _HALF * ph
                    prby = py + _HALF * ph
                    wx = jnp.maximum(
                        jnp.minimum(prbx, grbx) - jnp.maximum(pltx, gltx), 0.0)
                    wy = jnp.maximum(
                        jnp.minimum(prby, grby) - jnp.maximum(plty, glty), 0.0)
                    inter = wx * wy
                    ap = (prbx - pltx) * (prby - plty)
                    return inter / (ap + ag - inter + 1e-10)

                iou1 = iou(px1, py1, pw1, ph1)
                iou2 = iou(px2, py2, pw2, ph2)
                sel = iou2 > iou1
                rx = jnp.where(sel, px2, px1)
                ry = jnp.where(sel, py2, py1)
                rw = jnp.where(sel, pw2, pw1)
                rh = jnp.where(sel, ph2, ph1)
                rc = jnp.where(sel, pc2, pc1)
                ic = jnp.where(sel, pc1, pc2)
                miou = jnp.where(sel, iou2, iou1)

                dx = rx - gx
                dy = ry - gy
                coord = (dx * dx + dy * dy
                         + (rw + gw - 2.0 * _sqrt16(rw * gw))
                         + (rh + gh - 2.0 * _sqrt16(rh * gh)))
                dresp = rc - miou
                resp = dresp * dresp
                irr = ic * ic
                d4 = pc1 - g4
                d9 = pc2 - g9
                noobj = d4 * d4 + d9 * d9

                obj_term = 5.0 * coord + 2.0 * resp + irr + cls
                row = jnp.where(g4 > 0, obj_term, 0.5 * noobj)
                acc[...] += row

        pltpu.sync_copy(acc, out_hbm.at[wid])

    return body(p_flat, g_flat)


def kernel(pred_tensor, ground_truth):
    p_flat = pred_tensor.reshape(-1)
    g_flat = ground_truth.reshape(-1)
    partials = _sc_partials(p_flat, g_flat)
    return jnp.sum(partials) / _B


# speedup vs baseline: 56.8380x; 56.8380x over previous
import functools

import jax
import jax.numpy as jnp
from jax import lax
from jax.experimental import pallas as pl
from jax.experimental.pallas import tpu as pltpu
from jax.experimental.pallas import tpu_sc as plsc

_B = 512
_S = 14
_C = 30
_ROWS = _B * _S * _S
_NTILES = 32
_ROWS_PER_TILE = _ROWS // _NTILES
_CHUNK_ROWS = 784
_CHUNKS = _ROWS_PER_TILE // _CHUNK_ROWS
_CHUNK_WORDS = _CHUNK_ROWS * _C
_GROUPS = _CHUNK_ROWS // 16
_HALF = 0.5 * _S


def _sqrt16(x):
    i = plsc.bitcast(x, jnp.int32)
    i = jnp.int32(0x5F3759DF) - lax.shift_right_logical(i, 1)
    r = plsc.bitcast(i, jnp.float32)
    r = r * (1.5 - 0.5 * x * r * r)
    r = r * (1.5 - 0.5 * x * r * r)
    r = r * (1.5 - 0.5 * x * r * r)
    return x * r


def _sc_partials(p_flat, g_flat):
    mesh = plsc.VectorSubcoreMesh(core_axis_name="c", subcore_axis_name="s")

    @functools.partial(
        pl.kernel,
        mesh=mesh,
        out_type=jax.ShapeDtypeStruct((_NTILES, 16), jnp.float32),
        compiler_params=pltpu.CompilerParams(needs_layout_passes=False),
        scratch_types=[
            pltpu.VMEM((_CHUNK_WORDS,), jnp.float32),
            pltpu.VMEM((_CHUNK_WORDS,), jnp.float32),
            pltpu.VMEM((16,), jnp.float32),
        ],
    )
    def body(p_hbm, g_hbm, out_hbm, pbuf, gbuf, acc):
        wid = lax.axis_index("s") * 2 + lax.axis_index("c")
        acc[...] = jnp.zeros((16,), jnp.float32)
        row_off = lax.iota(jnp.int32, 16) * _C
        tile_base = wid * (_ROWS_PER_TILE * _C)

        @pl.loop(0, _CHUNKS)
        def _chunk(k):
            off = tile_base + k * _CHUNK_WORDS
            pltpu.sync_copy(p_hbm.at[pl.ds(off, _CHUNK_WORDS)], pbuf)
            pltpu.sync_copy(g_hbm.at[pl.ds(off, _CHUNK_WORDS)], gbuf)

            @pl.loop(0, _GROUPS)
            def _group(gi):
                idx0 = row_off + gi * (16 * _C)

                def lp(c):
                    return plsc.load_gather(pbuf, [idx0 + c])

                def lg(c):
                    return plsc.load_gather(gbuf, [idx0 + c])

                px1, py1, pw1, ph1, pc1 = lp(0), lp(1), lp(2), lp(3), lp(4)
                px2, py2, pw2, ph2, pc2 = lp(5), lp(6), lp(7), lp(8), lp(9)
                gx, gy, gw, gh, g4 = lg(0), lg(1), lg(2), lg(3), lg(4)
                g9 = lg(9)

                cls = jnp.zeros((16,), jnp.float32)
                for c in range(10, _C):
                    dcv = lp(c) - lg(c)
                    cls = cls + dcv * dcv

                gltx = gx - _HALF * gw
                grbx = gx + _HALF * gw
                glty = gy - _HALF * gh
                grby = gy + _HALF * gh
                ag = (grbx - gltx) * (grby - glty)

                def iou(px, py, pw, ph):
                    pltx = px - _HALF * pw
                    prbx = px + _HALF * pw
                    plty = py - _HALF * ph
                    prby = py + _HALF * ph
                    wx = jnp.maximum(
                        jnp.minimum(prbx, grbx) - jnp.maximum(pltx, gltx), 0.0)
                    wy = jnp.maximum(
                        jnp.minimum(prby, grby) - jnp.maximum(plty, glty), 0.0)
                    inter = wx * wy
                    ap = (prbx - pltx) * (prby - plty)
                    return inter / (ap + ag - inter + 1e-10)

                iou1 = iou(px1, py1, pw1, ph1)
                iou2 = iou(px2, py2, pw2, ph2)
                sel = iou2 > iou1
                rx = jnp.where(sel, px2, px1)
                ry = jnp.where(sel, py2, py1)
                rw = jnp.where(sel, pw2, pw1)
                rh = jnp.where(sel, ph2, ph1)
                rc = jnp.where(sel, pc2, pc1)
                ic = jnp.where(sel, pc1, pc2)
                miou = jnp.where(sel, iou2, iou1)

                dx = rx - gx
                dy = ry - gy
                coord = (dx * dx + dy * dy
                         + (rw + gw - 2.0 * _sqrt16(rw * gw))
                         + (rh + gh - 2.0 * _sqrt16(rh * gh)))
                dresp = rc - miou
                resp = dresp * dresp
                irr = ic * ic
                d4 = pc1 - g4
                d9 = pc2 - g9
                noobj = d4 * d4 + d9 * d9

                obj_term = 5.0 * coord + 2.0 * resp + irr + cls
                row = jnp.where(g4 > 0, obj_term, 0.5 * noobj)
                acc[...] += row

        pltpu.sync_copy(acc, out_hbm.at[wid])

    return body(p_flat, g_flat)


def kernel(pred_tensor, ground_truth):
    p_flat = pred_tensor.reshape(-1)
    g_flat = ground_truth.reshape(-1)
    partials = _sc_partials(p_flat, g_flat)
    return jnp.sum(partials) / _B
